# Initial kernel scaffold; baseline (speedup 1.0000x reference)
import jax
import jax.numpy as jnp
from jax.experimental import pallas as pl


def kernel(x, edge_index, edge_attr, W1, b1, W2, b2, W3, b3):
    raise NotImplementedError("write your pallas kernel here")


# speedup vs baseline: 6.2630x; 6.2630x over previous
import functools

import jax
import jax.numpy as jnp
from jax import lax
from jax.experimental import pallas as pl
from jax.experimental.pallas import tpu as pltpu
from jax.experimental.pallas import tpu_sc as plsc

N_NODES = 10000
N_EDGES = 320000
NODE = 128
EDGE_F = 16
HID = 32
LANES = 16

NC, NS = 2, 16
NW = NC * NS
PER_W = N_EDGES // NW
BATCH = 40
K = 5
SUP = BATCH * K
NSUP = PER_W // SUP
RS = SUP // 8
GCOL = 8 * HID
OCOL = GCOL + 8 * EDGE_F
ROWS_W = PER_W // 8

_F32 = jnp.float32
_HIGH = lax.Precision.HIGHEST


def _dot(a, b):
    return lax.dot_general(a, b, (((1,), (0,)), ((), ())),
                           precision=_HIGH, preferred_element_type=_F32)


P_BLK = 1000


def _proj_body(x_ref, w1_ref, t_ref, sx_ref):
    xb = x_ref[...]
    p1 = _dot(xb, w1_ref[0:NODE, :])
    p2 = _dot(xb, w1_ref[NODE:2 * NODE, :])
    t_ref[...] = jnp.concatenate(
        [p1, p2, jnp.zeros((P_BLK, NODE - 2 * HID), _F32)], axis=1)

    @pl.when(pl.program_id(0) == 0)
    def _init():
        sx_ref[...] = jnp.zeros_like(sx_ref)

    sx_ref[...] += jnp.sum(xb, axis=0, keepdims=True)


_proj_call = pl.pallas_call(
    _proj_body,
    grid=(N_NODES // P_BLK,),
    in_specs=[
        pl.BlockSpec((P_BLK, NODE), lambda i: (i, 0)),
        pl.BlockSpec((2 * NODE, HID), lambda i: (0, 0)),
    ],
    out_specs=[
        pl.BlockSpec((P_BLK, NODE), lambda i: (i, 0)),
        pl.BlockSpec((1, NODE), lambda i: (0, 0)),
    ],
    out_shape=[
        jax.ShapeDtypeStruct((N_NODES, NODE), _F32),
        jax.ShapeDtypeStruct((1, NODE), _F32),
    ],
)


SUPW = RS * NODE
TOT1D = NW * ROWS_W * NODE


@functools.cache
def _make_sc_gather_add():
    mesh = plsc.VectorSubcoreMesh(
        core_axis_name="c", subcore_axis_name="s",
        num_cores=NC, num_subcores=NS)
    return functools.partial(
        pl.kernel,
        out_type=[
            jax.ShapeDtypeStruct((TOT1D,), _F32),
            jax.ShapeDtypeStruct((TOT1D,), _F32),
            jax.ShapeDtypeStruct((TOT1D,), _F32),
        ],
        mesh=mesh,
        scratch_types=[
            pltpu.VMEM((N_EDGES // NW,), jnp.int32),
            pltpu.VMEM((N_EDGES // NW,), jnp.int32),
            pltpu.VMEM((K, BATCH, NODE), _F32),
            pltpu.VMEM((K, BATCH, NODE), _F32),
            pltpu.VMEM((SUP, EDGE_F), _F32),
            pltpu.VMEM((SUPW,), _F32),
            pltpu.VMEM((SUPW,), _F32),
            pltpu.VMEM((SUPW,), _F32),
            pltpu.SemaphoreType.DMA,
        ],
    )(_sc_body)


def _sc_body(t_hbm, ii_hbm, jj_hbm, ea_hbm, ga_hbm, gb_hbm, oe_hbm,
             ii_v, jj_v, a_v, b_v, ea_v, oa_v, ob_v, oe_v, sem):
    cid = lax.axis_index("c")
    sid = lax.axis_index("s")
    wid = sid * NC + cid
    pltpu.sync_copy(ii_hbm.at[pl.ds(wid * PER_W, PER_W)], ii_v)
    pltpu.sync_copy(jj_hbm.at[pl.ds(wid * PER_W, PER_W)], jj_v)

    def superstep(ss, carry):
        e0 = ss * SUP
        hs = [pltpu.async_copy(ea_hbm.at[wid, pl.ds(e0, SUP)], ea_v, sem)]
        for k in range(K):
            isl = pl.ds((ss * K + k) * BATCH, BATCH)
            hs.append(pltpu.async_copy(
                t_hbm.at[ii_v.at[isl]], a_v.at[k], sem))
            isl = pl.ds((ss * K + k) * BATCH, BATCH)
            hs.append(pltpu.async_copy(
                t_hbm.at[jj_v.at[isl]], b_v.at[k], sem))
        for h in hs:
            h.wait()

        def row(g, c2):
            k = g // (BATCH // 8)
            gg = g % (BATCH // 8)
            for q in range(8):
                r = gg * 8 + q
                o_v = oa_v if q < 4 else ob_v
                qq = q % 4
                for half in range(2):
                    sa = pl.ds(half * LANES, LANES)
                    sb = pl.ds(HID + half * LANES, LANES)
                    so = pl.ds(g * NODE + qq * HID + half * LANES, LANES)
                    o_v[so] = a_v[k, r, sa] + b_v[k, r, sb]
                se = pl.ds(g * NODE + q * EDGE_F, LANES)
                oe_v[se] = ea_v[k * BATCH + r, 0:LANES]
            return c2

        lax.fori_loop(0, RS, row, 0)
        out0 = (wid * ROWS_W + ss * RS) * NODE
        pltpu.sync_copy(oa_v, ga_hbm.at[pl.ds(out0, SUPW)])
        pltpu.sync_copy(ob_v, gb_hbm.at[pl.ds(out0, SUPW)])
        pltpu.sync_copy(oe_v, oe_hbm.at[pl.ds(out0, SUPW)])
        return carry

    lax.fori_loop(0, NSUP, superstep, 0)


R_BLK = 2000


def _reduce_body(ga_ref, gb_ref, oe_ref, wa_ref, wb_ref, b1p_ref, sx_ref,
                 w2_ref, b2_ref, w3_ref, b3_ref, v_ref, acca_ref, accb_ref):
    i = pl.program_id(0)

    @pl.when(i == 0)
    def _init():
        acca_ref[...] = jnp.zeros_like(acca_ref)
        accb_ref[...] = jnp.zeros_like(accb_ref)

    ea = oe_ref[...]
    pre_a = ga_ref[...] + _dot(ea, wa_ref[...]) + b1p_ref[...]
    pre_b = gb_ref[...] + _dot(ea, wb_ref[...]) + b1p_ref[...]
    acca_ref[...] += jnp.sum(jnp.maximum(pre_a, 0.0), axis=0, keepdims=True)
    accb_ref[...] += jnp.sum(jnp.maximum(pre_b, 0.0), axis=0, keepdims=True)

    @pl.when(i == pl.num_programs(0) - 1)
    def _finish():
        a = acca_ref[...] + accb_ref[...]
        sum_h = a[:, 0:HID]
        for q in range(1, 4):
            sum_h = sum_h + a[:, q * HID:(q + 1) * HID]
        msg = _dot(sum_h, w2_ref[...]) + N_EDGES * b2_ref[...]
        w3 = w3_ref[...]
        t1 = _dot(sx_ref[...], w3[0:NODE, :])
        t2 = _dot(msg, w3[NODE:NODE + HID, :])
        v_ref[...] = t1 + t2 + b3_ref[...]


_reduce_call = pl.pallas_call(
    _reduce_body,
    grid=(NW * ROWS_W // R_BLK,),
    in_specs=[
        pl.BlockSpec((R_BLK, NODE), lambda i: (i, 0)),
        pl.BlockSpec((R_BLK, NODE), lambda i: (i, 0)),
        pl.BlockSpec((R_BLK, NODE), lambda i: (i, 0)),
        pl.BlockSpec((NODE, NODE), lambda i: (0, 0)),
        pl.BlockSpec((NODE, NODE), lambda i: (0, 0)),
        pl.BlockSpec((1, NODE), lambda i: (0, 0)),
        pl.BlockSpec((1, NODE), lambda i: (0, 0)),
        pl.BlockSpec((HID, HID), lambda i: (0, 0)),
        pl.BlockSpec((1, HID), lambda i: (0, 0)),
        pl.BlockSpec((NODE + HID, 1), lambda i: (0, 0)),
        pl.BlockSpec((1, 1), lambda i: (0, 0)),
    ],
    out_specs=pl.BlockSpec((1, 1), lambda i: (0, 0)),
    out_shape=jax.ShapeDtypeStruct((1, 1), _F32),
    scratch_shapes=[pltpu.VMEM((1, NODE), _F32), pltpu.VMEM((1, NODE), _F32)],
)


def kernel(x, edge_index, edge_attr, W1, b1, W2, b2, W3, b3):
    ei = edge_index.astype(jnp.int32)
    ii = ei[0].reshape(N_EDGES)
    jj = ei[1].reshape(N_EDGES)
    ea = edge_attr.reshape(NW, PER_W, EDGE_F)

    t, sx = _proj_call(x, W1[:2 * NODE])
    ga, gb, oe = _make_sc_gather_add()(t, ii, jj, ea)
    ga = ga.reshape(NW * ROWS_W, NODE)
    gb = gb.reshape(NW * ROWS_W, NODE)
    oe = oe.reshape(NW * ROWS_W, NODE)

    eye8 = jnp.eye(8, dtype=_F32)
    wa = jnp.kron(eye8[:, 0:4], W1[2 * NODE:])
    wb = jnp.kron(eye8[:, 4:8], W1[2 * NODE:])
    b1p = jnp.tile(b1, 4).reshape(1, NODE)
    v = _reduce_call(ga, gb, oe, wa, wb, b1p, sx, W2, b2.reshape(1, HID),
                     W3, b3.reshape(1, 1))
    return v.reshape(1)


# speedup vs baseline: 9.0949x; 1.4522x over previous
import functools

import jax
import jax.numpy as jnp
from jax import lax
from jax.experimental import pallas as pl
from jax.experimental.pallas import tpu as pltpu
from jax.experimental.pallas import tpu_sc as plsc

N_NODES = 10000
N_EDGES = 320000
NODE = 128
EDGE_F = 16
HID = 32
LANES = 16

NC, NS = 2, 16
NW = NC * NS
PER_W = N_EDGES // NW
SUP = 80
NSUP = PER_W // SUP
RS = SUP // 8
ROWS_W = PER_W // 8

_F32 = jnp.float32
_HIGH = lax.Precision.HIGHEST


def _dot(a, b):
    return lax.dot_general(a, b, (((1,), (0,)), ((), ())),
                           precision=_HIGH, preferred_element_type=_F32)


P_BLK = 1000


def _proj_body(x_ref, w1_ref, t_ref, sx_ref):
    xb = x_ref[...]
    p1 = _dot(xb, w1_ref[0:NODE, :])
    p2 = _dot(xb, w1_ref[NODE:2 * NODE, :])
    t_ref[...] = jnp.concatenate(
        [p1, p2, jnp.zeros((P_BLK, NODE - 2 * HID), _F32)], axis=1)

    @pl.when(pl.program_id(0) == 0)
    def _init():
        sx_ref[...] = jnp.zeros_like(sx_ref)

    sx_ref[...] += jnp.sum(xb, axis=0, keepdims=True)


_proj_call = pl.pallas_call(
    _proj_body,
    grid=(N_NODES // P_BLK,),
    in_specs=[
        pl.BlockSpec((P_BLK, NODE), lambda i: (i, 0)),
        pl.BlockSpec((2 * NODE, HID), lambda i: (0, 0)),
    ],
    out_specs=[
        pl.BlockSpec((P_BLK, NODE), lambda i: (i, 0)),
        pl.BlockSpec((1, NODE), lambda i: (0, 0)),
    ],
    out_shape=[
        jax.ShapeDtypeStruct((N_NODES, NODE), _F32),
        jax.ShapeDtypeStruct((1, NODE), _F32),
    ],
)


SUPW = RS * NODE
TOT1D = NW * ROWS_W * NODE


@functools.cache
def _make_sc_gather_add():
    mesh = plsc.VectorSubcoreMesh(
        core_axis_name="c", subcore_axis_name="s",
        num_cores=NC, num_subcores=NS)
    return functools.partial(
        pl.kernel,
        out_type=[
            jax.ShapeDtypeStruct((TOT1D,), _F32),
            jax.ShapeDtypeStruct((TOT1D,), _F32),
            jax.ShapeDtypeStruct((TOT1D,), _F32),
        ],
        mesh=mesh,
        scratch_types=[
            pltpu.VMEM((PER_W,), jnp.int32),
            pltpu.VMEM((PER_W,), jnp.int32),
            pltpu.VMEM((2, SUP, NODE), _F32),
            pltpu.VMEM((2, SUP, NODE), _F32),
            pltpu.VMEM((2, SUP, EDGE_F), _F32),
            pltpu.VMEM((2, SUPW), _F32),
            pltpu.VMEM((2, SUPW), _F32),
            pltpu.VMEM((2, SUPW), _F32),
            pltpu.SemaphoreType.DMA,
            pltpu.SemaphoreType.DMA,
            pltpu.SemaphoreType.DMA,
            pltpu.SemaphoreType.DMA,
        ],
    )(_sc_body)


def _sc_body(t_hbm, ii_hbm, jj_hbm, ea_hbm, ga_hbm, gb_hbm, oe_hbm,
             ii_v, jj_v, a_v, b_v, ea_v, oa_v, ob_v, oe_v,
             sin0, sin1, sout0, sout1):
    cid = lax.axis_index("c")
    sid = lax.axis_index("s")
    wid = sid * NC + cid
    pltpu.sync_copy(ii_hbm.at[pl.ds(wid * PER_W, PER_W)], ii_v)
    pltpu.sync_copy(jj_hbm.at[pl.ds(wid * PER_W, PER_W)], jj_v)

    def in_copies(ss, pb, sem):
        e0 = ss * SUP
        return (
            pltpu.make_async_copy(
                ea_hbm.at[wid, pl.ds(e0, SUP)], ea_v.at[pb], sem),
            pltpu.make_async_copy(
                t_hbm.at[ii_v.at[pl.ds(e0, SUP)]], a_v.at[pb], sem),
            pltpu.make_async_copy(
                t_hbm.at[jj_v.at[pl.ds(e0, SUP)]], b_v.at[pb], sem),
        )

    def out_copies(ss, pb, sem):
        out0 = (wid * ROWS_W + ss * RS) * NODE
        osl = pl.ds(out0, SUPW)
        return (
            pltpu.make_async_copy(oa_v.at[pb], ga_hbm.at[osl], sem),
            pltpu.make_async_copy(ob_v.at[pb], gb_hbm.at[osl], sem),
            pltpu.make_async_copy(oe_v.at[pb], oe_hbm.at[osl], sem),
        )

    for c in in_copies(0, 0, sin0):
        c.start()

    def superstep(ss, carry):
        pb = lax.rem(ss, 2)

        @pl.when(ss + 1 < NSUP)
        def _prefetch():
            @pl.when(pb == 0)
            def _():
                for c in in_copies(ss + 1, 1, sin1):
                    c.start()

            @pl.when(pb == 1)
            def _():
                for c in in_copies(ss + 1, 0, sin0):
                    c.start()

        @pl.when(pb == 0)
        def _():
            for c in in_copies(ss, 0, sin0):
                c.wait()

        @pl.when(pb == 1)
        def _():
            for c in in_copies(ss, 1, sin1):
                c.wait()

        @pl.when(jnp.logical_and(ss >= 2, pb == 0))
        def _():
            for c in out_copies(ss, 0, sout0):
                c.wait()

        @pl.when(jnp.logical_and(ss >= 2, pb == 1))
        def _():
            for c in out_copies(ss, 1, sout1):
                c.wait()

        def row(g, c2):
            for q in range(8):
                r = g * 8 + q
                o_v = oa_v if q < 4 else ob_v
                qq = q % 4
                for half in range(2):
                    sa = pl.ds(half * LANES, LANES)
                    sb = pl.ds(HID + half * LANES, LANES)
                    so = pl.ds(g * NODE + qq * HID + half * LANES, LANES)
                    o_v[pb, so] = a_v[pb, r, sa] + b_v[pb, r, sb]
                se = pl.ds(g * NODE + q * EDGE_F, LANES)
                oe_v[pb, se] = ea_v[pb, r, 0:LANES]
            return c2

        lax.fori_loop(0, RS, row, 0)

        @pl.when(pb == 0)
        def _():
            for c in out_copies(ss, 0, sout0):
                c.start()

        @pl.when(pb == 1)
        def _():
            for c in out_copies(ss, 1, sout1):
                c.start()

        return carry

    lax.fori_loop(0, NSUP, superstep, 0)
    for c in out_copies(NSUP - 2, 0, sout0):
        c.wait()
    for c in out_copies(NSUP - 1, 1, sout1):
        c.wait()


R_BLK = 2000


def _reduce_body(ga_ref, gb_ref, oe_ref, wa_ref, wb_ref, b1p_ref, sx_ref,
                 w2_ref, b2_ref, w3_ref, b3_ref, v_ref, acca_ref, accb_ref):
    i = pl.program_id(0)

    @pl.when(i == 0)
    def _init():
        acca_ref[...] = jnp.zeros_like(acca_ref)
        accb_ref[...] = jnp.zeros_like(accb_ref)

    ea = oe_ref[...]
    pre_a = ga_ref[...] + _dot(ea, wa_ref[...]) + b1p_ref[...]
    pre_b = gb_ref[...] + _dot(ea, wb_ref[...]) + b1p_ref[...]
    acca_ref[...] += jnp.sum(jnp.maximum(pre_a, 0.0), axis=0, keepdims=True)
    accb_ref[...] += jnp.sum(jnp.maximum(pre_b, 0.0), axis=0, keepdims=True)

    @pl.when(i == pl.num_programs(0) - 1)
    def _finish():
        a = acca_ref[...] + accb_ref[...]
        sum_h = a[:, 0:HID]
        for q in range(1, 4):
            sum_h = sum_h + a[:, q * HID:(q + 1) * HID]
        msg = _dot(sum_h, w2_ref[...]) + N_EDGES * b2_ref[...]
        w3 = w3_ref[...]
        t1 = _dot(sx_ref[...], w3[0:NODE, :])
        t2 = _dot(msg, w3[NODE:NODE + HID, :])
        v_ref[...] = t1 + t2 + b3_ref[...]


_reduce_call = pl.pallas_call(
    _reduce_body,
    grid=(NW * ROWS_W // R_BLK,),
    in_specs=[
        pl.BlockSpec((R_BLK, NODE), lambda i: (i, 0)),
        pl.BlockSpec((R_BLK, NODE), lambda i: (i, 0)),
        pl.BlockSpec((R_BLK, NODE), lambda i: (i, 0)),
        pl.BlockSpec((NODE, NODE), lambda i: (0, 0)),
        pl.BlockSpec((NODE, NODE), lambda i: (0, 0)),
        pl.BlockSpec((1, NODE), lambda i: (0, 0)),
        pl.BlockSpec((1, NODE), lambda i: (0, 0)),
        pl.BlockSpec((HID, HID), lambda i: (0, 0)),
        pl.BlockSpec((1, HID), lambda i: (0, 0)),
        pl.BlockSpec((NODE + HID, 1), lambda i: (0, 0)),
        pl.BlockSpec((1, 1), lambda i: (0, 0)),
    ],
    out_specs=pl.BlockSpec((1, 1), lambda i: (0, 0)),
    out_shape=jax.ShapeDtypeStruct((1, 1), _F32),
    scratch_shapes=[pltpu.VMEM((1, NODE), _F32), pltpu.VMEM((1, NODE), _F32)],
)


def kernel(x, edge_index, edge_attr, W1, b1, W2, b2, W3, b3):
    ei = edge_index.astype(jnp.int32)
    ii = ei[0].reshape(N_EDGES)
    jj = ei[1].reshape(N_EDGES)
    ea = edge_attr.reshape(NW, PER_W, EDGE_F)

    t, sx = _proj_call(x, W1[:2 * NODE])
    ga, gb, oe = _make_sc_gather_add()(t, ii, jj, ea)
    ga = ga.reshape(NW * ROWS_W, NODE)
    gb = gb.reshape(NW * ROWS_W, NODE)
    oe = oe.reshape(NW * ROWS_W, NODE)

    eye8 = jnp.eye(8, dtype=_F32)
    wa = jnp.kron(eye8[:, 0:4], W1[2 * NODE:])
    wb = jnp.kron(eye8[:, 4:8], W1[2 * NODE:])
    b1p = jnp.tile(b1, 4).reshape(1, NODE)
    v = _reduce_call(ga, gb, oe, wa, wb, b1p, sx, W2, b2.reshape(1, HID),
                     W3, b3.reshape(1, 1))
    return v.reshape(1)


# speedup vs baseline: 9.6752x; 1.0638x over previous
import functools

import jax
import jax.numpy as jnp
from jax import lax
from jax.experimental import pallas as pl
from jax.experimental.pallas import tpu as pltpu
from jax.experimental.pallas import tpu_sc as plsc

N_NODES = 10000
N_EDGES = 320000
NODE = 128
EDGE_F = 16
HID = 32
LANES = 16

NC, NS = 2, 16
NW = NC * NS
PER_W = N_EDGES // NW
SUP = 80
NSUP = PER_W // SUP
RS = SUP // 8
ROWS_W = PER_W // 8

_F32 = jnp.float32
_HIGH = lax.Precision.HIGHEST


def _dot(a, b):
    return lax.dot_general(a, b, (((1,), (0,)), ((), ())),
                           precision=_HIGH, preferred_element_type=_F32)


def _dot16(a, b):
    return lax.dot_general(a, b, (((1,), (0,)), ((), ())),
                           preferred_element_type=_F32)


P_BLK = 1000


def _proj_body(x_ref, w1_ref, t_ref, sx_ref):
    xb = x_ref[...]
    xb16 = xb.astype(jnp.bfloat16)
    p1 = _dot16(xb16, w1_ref[0:NODE, :].astype(jnp.bfloat16))
    p2 = _dot16(xb16, w1_ref[NODE:2 * NODE, :].astype(jnp.bfloat16))
    t_ref[...] = jnp.concatenate(
        [p1, p2, jnp.zeros((P_BLK, NODE - 2 * HID), _F32)], axis=1)

    @pl.when(pl.program_id(0) == 0)
    def _init():
        sx_ref[...] = jnp.zeros_like(sx_ref)

    sx_ref[...] += jnp.sum(xb, axis=0, keepdims=True)


_proj_call = pl.pallas_call(
    _proj_body,
    grid=(N_NODES // P_BLK,),
    in_specs=[
        pl.BlockSpec((P_BLK, NODE), lambda i: (i, 0)),
        pl.BlockSpec((2 * NODE, HID), lambda i: (0, 0)),
    ],
    out_specs=[
        pl.BlockSpec((P_BLK, NODE), lambda i: (i, 0)),
        pl.BlockSpec((1, NODE), lambda i: (0, 0)),
    ],
    out_shape=[
        jax.ShapeDtypeStruct((N_NODES, NODE), _F32),
        jax.ShapeDtypeStruct((1, NODE), _F32),
    ],
)


SUPW = RS * NODE
TOT1D = NW * ROWS_W * NODE


@functools.cache
def _make_sc_gather_add():
    mesh = plsc.VectorSubcoreMesh(
        core_axis_name="c", subcore_axis_name="s",
        num_cores=NC, num_subcores=NS)
    return functools.partial(
        pl.kernel,
        out_type=[
            jax.ShapeDtypeStruct((TOT1D,), _F32),
            jax.ShapeDtypeStruct((TOT1D,), _F32),
            jax.ShapeDtypeStruct((TOT1D,), _F32),
        ],
        mesh=mesh,
        scratch_types=[
            pltpu.VMEM((PER_W,), jnp.int32),
            pltpu.VMEM((PER_W,), jnp.int32),
            pltpu.VMEM((2, SUP, NODE), _F32),
            pltpu.VMEM((2, SUP, NODE), _F32),
            pltpu.VMEM((2, SUP, EDGE_F), _F32),
            pltpu.VMEM((2, SUPW), _F32),
            pltpu.VMEM((2, SUPW), _F32),
            pltpu.VMEM((2, SUPW), _F32),
            pltpu.SemaphoreType.DMA,
            pltpu.SemaphoreType.DMA,
            pltpu.SemaphoreType.DMA,
            pltpu.SemaphoreType.DMA,
        ],
    )(_sc_body)


def _sc_body(t_hbm, ii_hbm, jj_hbm, ea_hbm, ga_hbm, gb_hbm, oe_hbm,
             ii_v, jj_v, a_v, b_v, ea_v, oa_v, ob_v, oe_v,
             sin0, sin1, sout0, sout1):
    cid = lax.axis_index("c")
    sid = lax.axis_index("s")
    wid = sid * NC + cid
    pltpu.sync_copy(ii_hbm.at[pl.ds(wid * PER_W, PER_W)], ii_v)
    pltpu.sync_copy(jj_hbm.at[pl.ds(wid * PER_W, PER_W)], jj_v)

    def in_copies(ss, pb, sem):
        e0 = ss * SUP
        return (
            pltpu.make_async_copy(
                ea_hbm.at[wid, pl.ds(e0, SUP)], ea_v.at[pb], sem),
            pltpu.make_async_copy(
                t_hbm.at[ii_v.at[pl.ds(e0, SUP)]], a_v.at[pb], sem),
            pltpu.make_async_copy(
                t_hbm.at[jj_v.at[pl.ds(e0, SUP)]], b_v.at[pb], sem),
        )

    def out_copies(ss, pb, sem):
        out0 = (wid * ROWS_W + ss * RS) * NODE
        osl = pl.ds(out0, SUPW)
        return (
            pltpu.make_async_copy(oa_v.at[pb], ga_hbm.at[osl], sem),
            pltpu.make_async_copy(ob_v.at[pb], gb_hbm.at[osl], sem),
            pltpu.make_async_copy(oe_v.at[pb], oe_hbm.at[osl], sem),
        )

    for c in in_copies(0, 0, sin0):
        c.start()

    def superstep(ss, carry):
        pb = lax.rem(ss, 2)

        @pl.when(ss + 1 < NSUP)
        def _prefetch():
            @pl.when(pb == 0)
            def _():
                for c in in_copies(ss + 1, 1, sin1):
                    c.start()

            @pl.when(pb == 1)
            def _():
                for c in in_copies(ss + 1, 0, sin0):
                    c.start()

        @pl.when(pb == 0)
        def _():
            for c in in_copies(ss, 0, sin0):
                c.wait()

        @pl.when(pb == 1)
        def _():
            for c in in_copies(ss, 1, sin1):
                c.wait()

        @pl.when(jnp.logical_and(ss >= 2, pb == 0))
        def _():
            for c in out_copies(ss, 0, sout0):
                c.wait()

        @pl.when(jnp.logical_and(ss >= 2, pb == 1))
        def _():
            for c in out_copies(ss, 1, sout1):
                c.wait()

        def row(g, c2):
            for q in range(8):
                r = g * 8 + q
                o_v = oa_v if q < 4 else ob_v
                qq = q % 4
                for half in range(2):
                    sa = pl.ds(half * LANES, LANES)
                    sb = pl.ds(HID + half * LANES, LANES)
                    so = pl.ds(g * NODE + qq * HID + half * LANES, LANES)
                    o_v[pb, so] = a_v[pb, r, sa] + b_v[pb, r, sb]
                se = pl.ds(g * NODE + q * EDGE_F, LANES)
                oe_v[pb, se] = ea_v[pb, r, 0:LANES]
            return c2

        lax.fori_loop(0, RS, row, 0)

        @pl.when(pb == 0)
        def _():
            for c in out_copies(ss, 0, sout0):
                c.start()

        @pl.when(pb == 1)
        def _():
            for c in out_copies(ss, 1, sout1):
                c.start()

        return carry

    lax.fori_loop(0, NSUP, superstep, 0)
    for c in out_copies(NSUP - 2, 0, sout0):
        c.wait()
    for c in out_copies(NSUP - 1, 1, sout1):
        c.wait()


R_BLK = 2000


def _reduce_body(ga_ref, gb_ref, oe_ref, wa_ref, wb_ref, b1p_ref, sx_ref,
                 w2_ref, b2_ref, w3_ref, b3_ref, v_ref, acca_ref, accb_ref):
    i = pl.program_id(0)

    @pl.when(i == 0)
    def _init():
        acca_ref[...] = jnp.zeros_like(acca_ref)
        accb_ref[...] = jnp.zeros_like(accb_ref)

    ea = oe_ref[...].astype(jnp.bfloat16)
    wa16 = wa_ref[...].astype(jnp.bfloat16)
    wb16 = wb_ref[...].astype(jnp.bfloat16)
    pre_a = ga_ref[...] + _dot16(ea, wa16) + b1p_ref[...]
    pre_b = gb_ref[...] + _dot16(ea, wb16) + b1p_ref[...]
    h_a = jnp.maximum(pre_a, 0.0).astype(jnp.bfloat16).astype(_F32)
    h_b = jnp.maximum(pre_b, 0.0).astype(jnp.bfloat16).astype(_F32)
    acca_ref[...] += jnp.sum(h_a, axis=0, keepdims=True)
    accb_ref[...] += jnp.sum(h_b, axis=0, keepdims=True)

    @pl.when(i == pl.num_programs(0) - 1)
    def _finish():
        a = acca_ref[...] + accb_ref[...]
        sum_h = a[:, 0:HID]
        for q in range(1, 4):
            sum_h = sum_h + a[:, q * HID:(q + 1) * HID]
        w2r = w2_ref[...].astype(jnp.bfloat16).astype(_F32)
        msg = _dot(sum_h, w2r) + N_EDGES * b2_ref[...]
        w3 = w3_ref[...]
        t1 = _dot(sx_ref[...], w3[0:NODE, :])
        t2 = _dot(msg, w3[NODE:NODE + HID, :])
        v_ref[...] = t1 + t2 + b3_ref[...]


_reduce_call = pl.pallas_call(
    _reduce_body,
    grid=(NW * ROWS_W // R_BLK,),
    in_specs=[
        pl.BlockSpec((R_BLK, NODE), lambda i: (i, 0)),
        pl.BlockSpec((R_BLK, NODE), lambda i: (i, 0)),
        pl.BlockSpec((R_BLK, NODE), lambda i: (i, 0)),
        pl.BlockSpec((NODE, NODE), lambda i: (0, 0)),
        pl.BlockSpec((NODE, NODE), lambda i: (0, 0)),
        pl.BlockSpec((1, NODE), lambda i: (0, 0)),
        pl.BlockSpec((1, NODE), lambda i: (0, 0)),
        pl.BlockSpec((HID, HID), lambda i: (0, 0)),
        pl.BlockSpec((1, HID), lambda i: (0, 0)),
        pl.BlockSpec((NODE + HID, 1), lambda i: (0, 0)),
        pl.BlockSpec((1, 1), lambda i: (0, 0)),
    ],
    out_specs=pl.BlockSpec((1, 1), lambda i: (0, 0)),
    out_shape=jax.ShapeDtypeStruct((1, 1), _F32),
    scratch_shapes=[pltpu.VMEM((1, NODE), _F32), pltpu.VMEM((1, NODE), _F32)],
)


def kernel(x, edge_index, edge_attr, W1, b1, W2, b2, W3, b3):
    ei = edge_index.astype(jnp.int32)
    ii = ei[0].reshape(N_EDGES)
    jj = ei[1].reshape(N_EDGES)
    ea = edge_attr.reshape(NW, PER_W, EDGE_F)

    t, sx = _proj_call(x, W1[:2 * NODE])
    ga, gb, oe = _make_sc_gather_add()(t, ii, jj, ea)
    ga = ga.reshape(NW * ROWS_W, NODE)
    gb = gb.reshape(NW * ROWS_W, NODE)
    oe = oe.reshape(NW * ROWS_W, NODE)

    eye8 = jnp.eye(8, dtype=_F32)
    wa = jnp.kron(eye8[:, 0:4], W1[2 * NODE:])
    wb = jnp.kron(eye8[:, 4:8], W1[2 * NODE:])
    b1p = jnp.tile(b1, 4).reshape(1, NODE)
    v = _reduce_call(ga, gb, oe, wa, wb, b1p, sx, W2, b2.reshape(1, HID),
                     W3, b3.reshape(1, 1))
    return v.reshape(1)


# speedup vs baseline: 9.7116x; 1.0038x over previous
import functools

import jax
import jax.numpy as jnp
from jax import lax
from jax.experimental import pallas as pl
from jax.experimental.pallas import tpu as pltpu
from jax.experimental.pallas import tpu_sc as plsc

N_NODES = 10000
N_EDGES = 320000
NODE = 128
EDGE_F = 16
HID = 32
LANES = 16

NC, NS = 2, 16
NW = NC * NS
PER_W = N_EDGES // NW
SUP = 80
NSUP = PER_W // SUP
RS = SUP // 8
ROWS_W = PER_W // 8

_F32 = jnp.float32
_HIGH = lax.Precision.HIGHEST


def _dot(a, b):
    return lax.dot_general(a, b, (((1,), (0,)), ((), ())),
                           precision=_HIGH, preferred_element_type=_F32)


def _dot16(a, b):
    return lax.dot_general(a, b, (((1,), (0,)), ((), ())),
                           preferred_element_type=_F32)


P_BLK = 1000


def _proj_body(x_ref, w1_ref, t_ref, sx_ref):
    xb = x_ref[...]
    xb16 = xb.astype(jnp.bfloat16)
    p1 = _dot16(xb16, w1_ref[0:NODE, :].astype(jnp.bfloat16))
    p2 = _dot16(xb16, w1_ref[NODE:2 * NODE, :].astype(jnp.bfloat16))
    t_ref[...] = jnp.concatenate(
        [p1, p2, jnp.zeros((P_BLK, NODE - 2 * HID), _F32)], axis=1)

    @pl.when(pl.program_id(0) == 0)
    def _init():
        sx_ref[...] = jnp.zeros_like(sx_ref)

    sx_ref[...] += jnp.sum(xb, axis=0, keepdims=True)


_proj_call = pl.pallas_call(
    _proj_body,
    grid=(N_NODES // P_BLK,),
    in_specs=[
        pl.BlockSpec((P_BLK, NODE), lambda i: (i, 0)),
        pl.BlockSpec((2 * NODE, HID), lambda i: (0, 0)),
    ],
    out_specs=[
        pl.BlockSpec((P_BLK, NODE), lambda i: (i, 0)),
        pl.BlockSpec((1, NODE), lambda i: (0, 0)),
    ],
    out_shape=[
        jax.ShapeDtypeStruct((N_NODES, NODE), _F32),
        jax.ShapeDtypeStruct((1, NODE), _F32),
    ],
)


SUPW = RS * NODE
TOT1D = NW * ROWS_W * NODE


@functools.cache
def _make_sc_gather_add():
    mesh = plsc.VectorSubcoreMesh(
        core_axis_name="c", subcore_axis_name="s",
        num_cores=NC, num_subcores=NS)
    return functools.partial(
        pl.kernel,
        out_type=[
            jax.ShapeDtypeStruct((TOT1D,), _F32),
            jax.ShapeDtypeStruct((TOT1D,), _F32),
            jax.ShapeDtypeStruct((TOT1D,), _F32),
        ],
        mesh=mesh,
        scratch_types=[
            pltpu.VMEM((PER_W,), jnp.int32),
            pltpu.VMEM((PER_W,), jnp.int32),
            pltpu.VMEM((2, SUP, NODE), _F32),
            pltpu.VMEM((2, SUP, NODE), _F32),
            pltpu.VMEM((2, SUP, EDGE_F), _F32),
            pltpu.VMEM((2, SUPW), _F32),
            pltpu.VMEM((2, SUPW), _F32),
            pltpu.VMEM((2, SUPW), _F32),
            pltpu.SemaphoreType.DMA,
            pltpu.SemaphoreType.DMA,
            pltpu.SemaphoreType.DMA,
            pltpu.SemaphoreType.DMA,
        ],
    )(_sc_body)


def _sc_body(t_hbm, ij_hbm, ea_hbm, ga_hbm, gb_hbm, oe_hbm,
             ii_v, jj_v, a_v, b_v, ea_v, oa_v, ob_v, oe_v,
             sin0, sin1, sout0, sout1):
    cid = lax.axis_index("c")
    sid = lax.axis_index("s")
    wid = sid * NC + cid
    pltpu.sync_copy(ij_hbm.at[pl.ds(wid * PER_W, PER_W)], ii_v)
    pltpu.sync_copy(
        ij_hbm.at[pl.ds(N_EDGES + wid * PER_W, PER_W)], jj_v)

    def in_copies(ss, pb, sem):
        e0 = ss * SUP
        return (
            pltpu.make_async_copy(
                ea_hbm.at[wid, pl.ds(e0, SUP)], ea_v.at[pb], sem),
            pltpu.make_async_copy(
                t_hbm.at[ii_v.at[pl.ds(e0, SUP)]], a_v.at[pb], sem),
            pltpu.make_async_copy(
                t_hbm.at[jj_v.at[pl.ds(e0, SUP)]], b_v.at[pb], sem),
        )

    def out_copies(ss, pb, sem):
        out0 = (wid * ROWS_W + ss * RS) * NODE
        osl = pl.ds(out0, SUPW)
        return (
            pltpu.make_async_copy(oa_v.at[pb], ga_hbm.at[osl], sem),
            pltpu.make_async_copy(ob_v.at[pb], gb_hbm.at[osl], sem),
            pltpu.make_async_copy(oe_v.at[pb], oe_hbm.at[osl], sem),
        )

    for c in in_copies(0, 0, sin0):
        c.start()

    def superstep(ss, carry):
        pb = lax.rem(ss, 2)

        @pl.when(ss + 1 < NSUP)
        def _prefetch():
            @pl.when(pb == 0)
            def _():
                for c in in_copies(ss + 1, 1, sin1):
                    c.start()

            @pl.when(pb == 1)
            def _():
                for c in in_copies(ss + 1, 0, sin0):
                    c.start()

        @pl.when(pb == 0)
        def _():
            for c in in_copies(ss, 0, sin0):
                c.wait()

        @pl.when(pb == 1)
        def _():
            for c in in_copies(ss, 1, sin1):
                c.wait()

        @pl.when(jnp.logical_and(ss >= 2, pb == 0))
        def _():
            for c in out_copies(ss, 0, sout0):
                c.wait()

        @pl.when(jnp.logical_and(ss >= 2, pb == 1))
        def _():
            for c in out_copies(ss, 1, sout1):
                c.wait()

        def row(g, c2):
            for q in range(8):
                r = g * 8 + q
                o_v = oa_v if q < 4 else ob_v
                qq = q % 4
                for half in range(2):
                    sa = pl.ds(half * LANES, LANES)
                    sb = pl.ds(HID + half * LANES, LANES)
                    so = pl.ds(g * NODE + qq * HID + half * LANES, LANES)
                    o_v[pb, so] = a_v[pb, r, sa] + b_v[pb, r, sb]
                se = pl.ds(g * NODE + q * EDGE_F, LANES)
                oe_v[pb, se] = ea_v[pb, r, 0:LANES]
            return c2

        lax.fori_loop(0, RS, row, 0)

        @pl.when(pb == 0)
        def _():
            for c in out_copies(ss, 0, sout0):
                c.start()

        @pl.when(pb == 1)
        def _():
            for c in out_copies(ss, 1, sout1):
                c.start()

        return carry

    lax.fori_loop(0, NSUP, superstep, 0)
    for c in out_copies(NSUP - 2, 0, sout0):
        c.wait()
    for c in out_copies(NSUP - 1, 1, sout1):
        c.wait()


R_BLK = 2000


def _reduce_body(ga_ref, gb_ref, oe_ref, wa_ref, wb_ref, b1p_ref, sx_ref,
                 w2_ref, b2_ref, w3_ref, b3_ref, v_ref, acca_ref, accb_ref):
    i = pl.program_id(0)

    @pl.when(i == 0)
    def _init():
        acca_ref[...] = jnp.zeros_like(acca_ref)
        accb_ref[...] = jnp.zeros_like(accb_ref)

    ea = oe_ref[...].astype(jnp.bfloat16)
    wa16 = wa_ref[...].astype(jnp.bfloat16)
    wb16 = wb_ref[...].astype(jnp.bfloat16)
    pre_a = ga_ref[...] + _dot16(ea, wa16) + b1p_ref[...]
    pre_b = gb_ref[...] + _dot16(ea, wb16) + b1p_ref[...]
    h_a = jnp.maximum(pre_a, 0.0).astype(jnp.bfloat16).astype(_F32)
    h_b = jnp.maximum(pre_b, 0.0).astype(jnp.bfloat16).astype(_F32)
    acca_ref[...] += jnp.sum(h_a, axis=0, keepdims=True)
    accb_ref[...] += jnp.sum(h_b, axis=0, keepdims=True)

    @pl.when(i == pl.num_programs(0) - 1)
    def _finish():
        a = acca_ref[...] + accb_ref[...]
        sum_h = a[:, 0:HID]
        for q in range(1, 4):
            sum_h = sum_h + a[:, q * HID:(q + 1) * HID]
        w2r = w2_ref[...].astype(jnp.bfloat16).astype(_F32)
        msg = _dot(sum_h, w2r) + N_EDGES * b2_ref[...]
        w3 = w3_ref[...]
        t1 = _dot(sx_ref[...], w3[0:NODE, :])
        t2 = _dot(msg, w3[NODE:NODE + HID, :])
        v_ref[...] = t1 + t2 + b3_ref[...]


_reduce_call = pl.pallas_call(
    _reduce_body,
    grid=(NW * ROWS_W // R_BLK,),
    in_specs=[
        pl.BlockSpec((R_BLK, NODE), lambda i: (i, 0)),
        pl.BlockSpec((R_BLK, NODE), lambda i: (i, 0)),
        pl.BlockSpec((R_BLK, NODE), lambda i: (i, 0)),
        pl.BlockSpec((NODE, NODE), lambda i: (0, 0)),
        pl.BlockSpec((NODE, NODE), lambda i: (0, 0)),
        pl.BlockSpec((1, NODE), lambda i: (0, 0)),
        pl.BlockSpec((1, NODE), lambda i: (0, 0)),
        pl.BlockSpec((HID, HID), lambda i: (0, 0)),
        pl.BlockSpec((1, HID), lambda i: (0, 0)),
        pl.BlockSpec((NODE + HID, 1), lambda i: (0, 0)),
        pl.BlockSpec((1, 1), lambda i: (0, 0)),
    ],
    out_specs=pl.BlockSpec((1, 1), lambda i: (0, 0)),
    out_shape=jax.ShapeDtypeStruct((1, 1), _F32),
    scratch_shapes=[pltpu.VMEM((1, NODE), _F32), pltpu.VMEM((1, NODE), _F32)],
)


def kernel(x, edge_index, edge_attr, W1, b1, W2, b2, W3, b3):
    ij = edge_index.astype(jnp.int32).reshape(2 * N_EDGES)
    ea = edge_attr.reshape(NW, PER_W, EDGE_F)

    t, sx = _proj_call(x, W1[:2 * NODE])
    ga, gb, oe = _make_sc_gather_add()(t, ij, ea)
    ga = ga.reshape(NW * ROWS_W, NODE)
    gb = gb.reshape(NW * ROWS_W, NODE)
    oe = oe.reshape(NW * ROWS_W, NODE)

    eye8 = jnp.eye(8, dtype=_F32)
    wa = jnp.kron(eye8[:, 0:4], W1[2 * NODE:])
    wb = jnp.kron(eye8[:, 4:8], W1[2 * NODE:])
    b1p = jnp.tile(b1, 4).reshape(1, NODE)
    v = _reduce_call(ga, gb, oe, wa, wb, b1p, sx, W2, b2.reshape(1, HID),
                     W3, b3.reshape(1, 1))
    return v.reshape(1)
